# Initial kernel scaffold; baseline (speedup 1.0000x reference)
import jax
import jax.numpy as jnp
from jax.experimental import pallas as pl


def kernel(x, edge_index, substring_embed, batch, W1, a_src1, a_dst1, b1, W2, a_src2, a_dst2, b2, Wo, bo):
    raise NotImplementedError("write your pallas kernel here")


# speedup vs baseline: 17.9156x; 17.9156x over previous
import functools

import jax
import jax.numpy as jnp
from jax import lax
from jax.experimental import pallas as pl
from jax.experimental.pallas import tpu as pltpu
from jax.experimental.pallas import tpu_sc as plsc

N = 10000
E = 320000
IN = 128
COND = 128
HID = 128
H1 = 4
NGRAPH = 64

NW = 32
CHUNK = 128
EPW = 10112
EP = NW * EPW
NCHUNK = EPW // CHUNK
ZROWS = 40
DEN_PER_TILE = 1000
NWRITE = N // DEN_PER_TILE

BN = 400
GRID = N // BN


def _make_sc_gat(heads):
  mesh = plsc.VectorSubcoreMesh(core_axis_name="c", subcore_axis_name="s")

  @functools.partial(
      pl.kernel,
      out_type=(
          jax.ShapeDtypeStruct((2, N, heads * HID), jnp.float32),
          jax.ShapeDtypeStruct((2 * heads * N,), jnp.float32),
      ),
      mesh=mesh,
      compiler_params=pltpu.CompilerParams(needs_layout_passes=False),
      scratch_types=(
          pltpu.VMEM((CHUNK,), jnp.int32),
          pltpu.VMEM((CHUNK,), jnp.int32),
          pltpu.VMEM((CHUNK,), jnp.int32),
          pltpu.VMEM((CHUNK,), jnp.float32),
          pltpu.VMEM((CHUNK, HID), jnp.float32),
          pltpu.VMEM((N,), jnp.float32),
          pltpu.VMEM((N,), jnp.float32),
          pltpu.VMEM((ZROWS, HID), jnp.float32),
          pltpu.VMEM((DEN_PER_TILE,), jnp.float32),
          pltpu.VMEM((ZROWS, HID), jnp.float32),
          pltpu.VMEM((DEN_PER_TILE,), jnp.float32),
          pltpu.VMEM_SHARED((N, HID), jnp.float32),
          pltpu.VMEM_SHARED((N,), jnp.float32),
          pltpu.SemaphoreType.DMA,
      ),
  )
  def sc_gat(src_hbm, dst_hbm, tab_hbm, asrc_hbm, adst_hbm,
             acc_out, den_out,
             src_v, dst_v, gidx_v, alpha_v, rows_v, as_t, ad_t,
             zero2, zero1, stage2, stage1, acc_s, den_s, sem):
    c = lax.axis_index("c")
    s = lax.axis_index("s")
    wid = c * 16 + s
    base = wid * EPW

    zeros16 = jnp.zeros((16,), jnp.float32)
    for j in range(ZROWS):
      for i in range(HID // 16):
        zero2[j, pl.ds(i * 16, 16)] = zeros16
    for j in range(DEN_PER_TILE // 16):
      zero1[pl.ds(j * 16, 16)] = zeros16

    for h in range(heads):
      pltpu.sync_copy(asrc_hbm.at[pl.ds(h * N, N)], as_t)
      pltpu.sync_copy(adst_hbm.at[pl.ds(h * N, N)], ad_t)

      @pl.when(s < NWRITE)
      def _zero():
        for j in range(DEN_PER_TILE // ZROWS):
          pltpu.sync_copy(zero2, acc_s.at[pl.ds(s * DEN_PER_TILE + j * ZROWS, ZROWS), :])
        pltpu.sync_copy(zero1, den_s.at[pl.ds(s * DEN_PER_TILE, DEN_PER_TILE)])

      plsc.subcore_barrier()

      def chunk_body(k, carry):
        off = base + k * CHUNK
        pltpu.sync_copy(src_hbm.at[pl.ds(off, CHUNK)], src_v)
        pltpu.sync_copy(dst_hbm.at[pl.ds(off, CHUNK)], dst_v)
        for g in range(CHUNK // 16):
          sl = pl.ds(g * 16, 16)
          sv = src_v[sl]
          dv = dst_v[sl]
          asv = plsc.load_gather(as_t, [sv])
          adv = plsc.load_gather(ad_t, [dv])
          e = asv + adv
          e = jnp.maximum(e, 0.2 * e)
          a = jnp.exp(e)
          gi = off + g * 16 + lax.iota(jnp.int32, 16)
          a = jnp.where(gi < E, a, 0.0)
          alpha_v[sl] = a
          gidx_v[sl] = sv * heads + h
        pltpu.async_copy(tab_hbm.at[gidx_v], rows_v, sem).wait()

        def scale_body(j, carry2):
          aj = plsc.load_gather(alpha_v, [jnp.full((16,), j, jnp.int32)])
          for i in range(HID // 16):
            sl2 = pl.ds(i * 16, 16)
            rows_v[j, sl2] = rows_v[j, sl2] * aj
          return carry2

        lax.fori_loop(0, CHUNK, scale_body, 0)
        pltpu.sync_copy(rows_v, acc_s.at[dst_v], add=True)
        pltpu.sync_copy(alpha_v, den_s.at[dst_v], add=True)
        return carry

      lax.fori_loop(0, NCHUNK, chunk_body, 0)
      plsc.subcore_barrier()

      @pl.when(s < NWRITE)
      def _write():
        for j in range(DEN_PER_TILE // ZROWS):
          r0 = s * DEN_PER_TILE + j * ZROWS
          pltpu.sync_copy(acc_s.at[pl.ds(r0, ZROWS), :], stage2)
          pltpu.sync_copy(stage2,
                          acc_out.at[c, pl.ds(r0, ZROWS), pl.ds(h * HID, HID)])
        pltpu.sync_copy(den_s.at[pl.ds(s * DEN_PER_TILE, DEN_PER_TILE)], stage1)
        pltpu.sync_copy(stage1,
                        den_out.at[pl.ds((c * heads + h) * N + s * DEN_PER_TILE, DEN_PER_TILE)])

      plsc.subcore_barrier()

  return sc_gat


_sc_gat_l1 = _make_sc_gat(H1)
_sc_gat_l2 = _make_sc_gat(1)


def _tc1_body(x_ref, bf_ref, se_ref, w1a_ref, w1b_ref, asr_ref, adr_ref,
              h1_ref, as_ref, ad_ref):
  xb = x_ref[...]
  bf = bf_ref[...]
  graph_iota = lax.broadcasted_iota(jnp.int32, (1, NGRAPH), 1).astype(jnp.float32)
  onehot = (bf == graph_iota).astype(jnp.float32)
  p = jnp.dot(se_ref[...], w1b_ref[...], preferred_element_type=jnp.float32)
  h1 = (jnp.dot(xb, w1a_ref[...], preferred_element_type=jnp.float32)
        + jnp.dot(onehot, p, preferred_element_type=jnp.float32))
  h1_ref[...] = h1
  as_ref[...] = jnp.dot(h1, asr_ref[...], preferred_element_type=jnp.float32)
  ad_ref[...] = jnp.dot(h1, adr_ref[...], preferred_element_type=jnp.float32)


def _tc2_body(acc_ref, den_ref, h1_ref, as_ref, ad_ref, b1_ref, w2_ref,
              as2_ref, ad2_ref, h2_ref, s2_ref, d2_ref):
  acc = acc_ref[...]
  accsum = acc[0] + acc[1]
  den = den_ref[...]
  dsum = den[:, :, 0] + den[:, :, 1]
  e = as_ref[...] + ad_ref[...]
  aself = jnp.exp(jnp.maximum(e, 0.2 * e))
  dtot = dsum + aself
  h1b = h1_ref[...]
  aw = jnp.repeat(aself, HID, axis=1)
  dw = jnp.repeat(dtot, HID, axis=1)
  numer = accsum + aw * h1b
  out1 = jnp.maximum(numer / (dw + 1e-16) + b1_ref[...], 0.0)
  h2 = jnp.dot(out1, w2_ref[...], preferred_element_type=jnp.float32)
  h2_ref[...] = h2
  s2_ref[...] = jnp.dot(h2, as2_ref[...].T, preferred_element_type=jnp.float32)
  d2_ref[...] = jnp.dot(h2, ad2_ref[...].T, preferred_element_type=jnp.float32)


def _tc3_body(acc_ref, den_ref, h2_ref, as_ref, ad_ref, b2_ref, wo_ref,
              bo_ref, o_ref):
  acc = acc_ref[...]
  accsum = acc[0] + acc[1]
  den = den_ref[...]
  dsum = den[:, 0, 0] + den[:, 0, 1]
  e = as_ref[...][:, 0] + ad_ref[...][:, 0]
  aself = jnp.exp(jnp.maximum(e, 0.2 * e))
  dtot = (dsum + aself)[:, None]
  numer = accsum + aself[:, None] * h2_ref[...]
  out2 = jnp.maximum(numer / (dtot + 1e-16) + b2_ref[...], 0.0)
  o_ref[...] = jnp.dot(out2, wo_ref[...], preferred_element_type=jnp.float32) + bo_ref[...]


def _whole(shape):
  return pl.BlockSpec(shape, lambda i: tuple(0 for _ in shape))


_tc1 = pl.pallas_call(
    _tc1_body,
    grid=(GRID,),
    in_specs=[
        pl.BlockSpec((BN, IN), lambda i: (i, 0)),
        pl.BlockSpec((BN, 1), lambda i: (i, 0)),
        _whole((NGRAPH, COND)),
        _whole((IN, H1 * HID)),
        _whole((COND, H1 * HID)),
        _whole((H1 * HID, H1)),
        _whole((H1 * HID, H1)),
    ],
    out_specs=[
        pl.BlockSpec((BN, H1 * HID), lambda i: (i, 0)),
        pl.BlockSpec((BN, H1), lambda i: (i, 0)),
        pl.BlockSpec((BN, H1), lambda i: (i, 0)),
    ],
    out_shape=[
        jax.ShapeDtypeStruct((N, H1 * HID), jnp.float32),
        jax.ShapeDtypeStruct((N, H1), jnp.float32),
        jax.ShapeDtypeStruct((N, H1), jnp.float32),
    ],
)

_tc2 = pl.pallas_call(
    _tc2_body,
    grid=(GRID,),
    in_specs=[
        pl.BlockSpec((2, BN, H1 * HID), lambda i: (0, i, 0)),
        pl.BlockSpec((BN, H1, 2), lambda i: (i, 0, 0)),
        pl.BlockSpec((BN, H1 * HID), lambda i: (i, 0)),
        pl.BlockSpec((BN, H1), lambda i: (i, 0)),
        pl.BlockSpec((BN, H1), lambda i: (i, 0)),
        _whole((1, H1 * HID)),
        _whole((H1 * HID, HID)),
        _whole((1, HID)),
        _whole((1, HID)),
    ],
    out_specs=[
        pl.BlockSpec((BN, HID), lambda i: (i, 0)),
        pl.BlockSpec((BN, 1), lambda i: (i, 0)),
        pl.BlockSpec((BN, 1), lambda i: (i, 0)),
    ],
    out_shape=[
        jax.ShapeDtypeStruct((N, HID), jnp.float32),
        jax.ShapeDtypeStruct((N, 1), jnp.float32),
        jax.ShapeDtypeStruct((N, 1), jnp.float32),
    ],
)

_tc3 = pl.pallas_call(
    _tc3_body,
    grid=(GRID,),
    in_specs=[
        pl.BlockSpec((2, BN, HID), lambda i: (0, i, 0)),
        pl.BlockSpec((BN, 1, 2), lambda i: (i, 0, 0)),
        pl.BlockSpec((BN, HID), lambda i: (i, 0)),
        pl.BlockSpec((BN, 1), lambda i: (i, 0)),
        pl.BlockSpec((BN, 1), lambda i: (i, 0)),
        _whole((1, HID)),
        _whole((HID, 1)),
        _whole((1, 1)),
    ],
    out_specs=pl.BlockSpec((BN, 1), lambda i: (i, 0)),
    out_shape=jax.ShapeDtypeStruct((N, 1), jnp.float32),
)


def kernel(x, edge_index, substring_embed, batch,
           W1, a_src1, a_dst1, b1, W2, a_src2, a_dst2, b2, Wo, bo):
  src = edge_index[0]
  dst = edge_index[1]
  padn = EP - E
  src_p = jnp.pad(src, (0, padn))
  dst_p = jnp.pad(dst, (0, padn))
  batch_f = batch.astype(jnp.float32).reshape(N, 1)

  w1a = W1[:IN]
  w1b = W1[IN:]
  eye = jnp.eye(H1, dtype=jnp.float32)
  asr1 = (eye[:, None, :] * a_src1[:, :, None]).reshape(H1 * HID, H1)
  adr1 = (eye[:, None, :] * a_dst1[:, :, None]).reshape(H1 * HID, H1)

  h1, as1, ad1 = _tc1(x, batch_f, substring_embed, w1a, w1b, asr1, adr1)

  acc1, den1 = _sc_gat_l1(src_p, dst_p, h1.reshape(N * H1, HID),
                          as1.T.reshape(H1 * N), ad1.T.reshape(H1 * N))
  den1n = den1.reshape(2, H1, N).transpose(2, 1, 0)

  h2, as2, ad2 = _tc2(acc1, den1n, h1, as1, ad1,
                      b1.reshape(1, H1 * HID), W2, a_src2, a_dst2)

  acc2, den2 = _sc_gat_l2(src_p, dst_p, h2, as2.reshape(N),
                          ad2.reshape(N))
  den2n = den2.reshape(2, 1, N).transpose(2, 1, 0)

  out = _tc3(acc2, den2n, h2, as2, ad2,
             b2.reshape(1, HID), Wo, bo.reshape(1, 1))
  return out.reshape(N)
